# baseline (speedup 1.0000x reference)
import jax
import jax.numpy as jnp
import numpy as np
from jax.experimental import pallas as pl
from jax.experimental.pallas import tpu as pltpu

B, N, D, E = 4, 2048, 4096, 64
TN = 1024
NT = N // TN

ONE_BITS = int(np.float32(1.0).view(np.int32))
MASK_BASE = 1 << 30


def _route(c_ref, probs_ref, m_ref, mp_ref, mask_ref):
    t_idx = jax.lax.broadcasted_iota(jnp.int32, (B, N), 1)
    mask_ref[...] = jnp.zeros((B, N), jnp.int32)
    m_ref[...] = jnp.full((B, N), E, jnp.int32)

    def cond(carry):
        i, U = carry
        return jnp.logical_and(i < E, U > 0)

    def step(carry):
        i, U = carry
        mask = mask_ref[...]
        M = m_ref[...]
        j = E - 1 - i
        c_j = c_ref[j, 0]
        kj = jnp.floor(c_j * np.float32(N)).astype(jnp.int32)
        pr = probs_ref[pl.ds(j, 1)][0]
        s = pltpu.bitcast(pr, jnp.int32)
        key = jnp.where(mask != 0, MASK_BASE + t_idx, ONE_BITS - s)

        def count(pred):
            return jnp.sum(pred.astype(jnp.int32), axis=1, keepdims=True)

        v = jnp.zeros((B, 1), jnp.int32)
        G = jnp.zeros((B, 1), jnp.int32)
        cand = v + (1 << 30)
        c1 = count(key < cand)
        t1 = c1 < kj
        v = jnp.where(t1, cand, v)
        G = jnp.where(t1, c1, G)
        for bit in range(28, -1, -2):
            c1 = count(key < v + (1 << bit))
            c2 = count(key < v + (2 << bit))
            c3 = count(key < v + (3 << bit))
            t1 = c1 < kj
            t2 = c2 < kj
            t3 = c3 < kj
            m = t1.astype(jnp.int32) + t2.astype(jnp.int32) + t3.astype(jnp.int32)
            v = v + m * (1 << bit)
            G = jnp.where(t3, c3, jnp.where(t2, c2, jnp.where(t1, c1, G)))
        tie = key == v
        need = kj - G
        tau = jnp.zeros((B, 1), jnp.int32)
        for bit in range(10, -1, -2):
            c1 = count(tie & (t_idx < tau + (1 << bit)))
            c2 = count(tie & (t_idx < tau + (2 << bit)))
            c3 = count(tie & (t_idx < tau + (3 << bit)))
            m = (
                (c1 < need).astype(jnp.int32)
                + (c2 < need).astype(jnp.int32)
                + (c3 < need).astype(jnp.int32)
            )
            tau = tau + m * (1 << bit)

        chosen = (key < v) | (tie & (t_idx <= tau))
        chosen = jnp.logical_and(chosen, kj > 0)
        m_ref[...] = jnp.where(chosen, j, M)
        mask_ref[...] = jnp.where(chosen, 1, mask)
        return i + 1, jnp.maximum(U - kj, 0)

    i_final, _ = jax.lax.while_loop(cond, step, (jnp.int32(0), jnp.int32(N)))

    Jr = E - 1 - i_final
    M = m_ref[...]
    for j in range(E):
        kj = jnp.floor(c_ref[j, 0] * np.float32(N)).astype(jnp.int32)
        cond_j = jnp.logical_and(t_idx < kj, j <= Jr)
        M = jnp.minimum(M, jnp.where(cond_j, j, E))
    M = jnp.where(M == E, 0, M)
    m_ref[...] = M
    Mp = jnp.zeros((B, N), jnp.float32)
    for j in range(E):
        Mp = jnp.where(M == j, probs_ref[j], Mp)
    mp_ref[...] = Mp


def _fused_body(c_ref, x_ref, w_ref, m_ref, mp_ref, probs_ref, mask_ref):
    b = pl.program_id(0)
    t = pl.program_id(1)
    off = pl.multiple_of(t * TN, TN)
    w = w_ref[...]
    xb = x_ref[0]
    logits = jax.lax.dot_general(
        w, xb, (((1,), (1,)), ((), ())), preferred_element_type=jnp.float32
    )
    mx = jnp.max(logits, axis=0, keepdims=True)
    e = jnp.exp(logits - mx)
    p = e / jnp.sum(e, axis=0, keepdims=True)
    probs_ref[:, b, pl.ds(off, TN)] = p

    @pl.when(jnp.logical_and(b == B - 1, t == NT - 1))
    def _():
        _route(c_ref, probs_ref, m_ref, mp_ref, mask_ref)


def kernel(x, c, W):
    M, Mp = pl.pallas_call(
        _fused_body,
        grid=(B, NT),
        in_specs=[
            pl.BlockSpec(memory_space=pltpu.MemorySpace.SMEM),
            pl.BlockSpec((1, TN, D), lambda b, t: (b, t, 0)),
            pl.BlockSpec((E, D), lambda b, t: (0, 0)),
        ],
        out_specs=[
            pl.BlockSpec((B, N), lambda b, t: (0, 0)),
            pl.BlockSpec((B, N), lambda b, t: (0, 0)),
        ],
        out_shape=[
            jax.ShapeDtypeStruct((B, N), jnp.int32),
            jax.ShapeDtypeStruct((B, N), jnp.float32),
        ],
        scratch_shapes=[
            pltpu.VMEM((E, B, N), jnp.float32),
            pltpu.VMEM((B, N), jnp.int32),
        ],
    )(c.reshape(E, 1), x, W)
    return M, Mp


# speedup vs baseline: 1.0154x; 1.0154x over previous
import jax
import jax.numpy as jnp
import numpy as np
from jax.experimental import pallas as pl
from jax.experimental.pallas import tpu as pltpu

B, N, D, E = 4, 2048, 4096, 64
TN = 512
NT = N // TN

ONE_BITS = int(np.float32(1.0).view(np.int32))
MASK_BASE = 1 << 30


def _route(c_ref, probs_ref, m_ref, mp_ref, mask_ref):
    t_idx = jax.lax.broadcasted_iota(jnp.int32, (B, N), 1)
    mask_ref[...] = jnp.zeros((B, N), jnp.int32)
    m_ref[...] = jnp.full((B, N), E, jnp.int32)

    def cond(carry):
        i, U = carry
        return jnp.logical_and(i < E, U > 0)

    def step(carry):
        i, U = carry
        mask = mask_ref[...]
        M = m_ref[...]
        j = E - 1 - i
        c_j = c_ref[j, 0]
        kj = jnp.floor(c_j * np.float32(N)).astype(jnp.int32)
        pr = probs_ref[pl.ds(j, 1)][0]
        s = pltpu.bitcast(pr, jnp.int32)
        key = jnp.where(mask != 0, MASK_BASE + t_idx, ONE_BITS - s)

        def count(pred):
            return jnp.sum(pred.astype(jnp.int32), axis=1, keepdims=True)

        v = jnp.zeros((B, 1), jnp.int32)
        G = jnp.zeros((B, 1), jnp.int32)
        cand = v + (1 << 30)
        c1 = count(key < cand)
        t1 = c1 < kj
        v = jnp.where(t1, cand, v)
        G = jnp.where(t1, c1, G)
        for bit in range(27, -1, -3):
            cs = [count(key < v + (q << bit)) for q in range(1, 8)]
            ts = [cq < kj for cq in cs]
            m = sum(tq.astype(jnp.int32) for tq in ts)
            v = v + m * (1 << bit)
            for cq, tq in zip(cs, ts):
                G = jnp.where(tq, cq, G)
        tie = key == v
        need = kj - G
        tau = jnp.zeros((B, 1), jnp.int32)
        for bit in range(9, -1, -3):
            cs = [count(tie & (t_idx < tau + (q << bit))) for q in range(1, 8)]
            m = sum((cq < need).astype(jnp.int32) for cq in cs)
            tau = tau + m * (1 << bit)

        chosen = (key < v) | (tie & (t_idx <= tau))
        chosen = jnp.logical_and(chosen, kj > 0)
        m_ref[...] = jnp.where(chosen, j, M)
        mask_ref[...] = jnp.where(chosen, 1, mask)
        return i + 1, jnp.maximum(U - kj, 0)

    i_final, _ = jax.lax.while_loop(cond, step, (jnp.int32(0), jnp.int32(N)))

    Jr = E - 1 - i_final
    M = m_ref[...]
    for j in range(E):
        kj = jnp.floor(c_ref[j, 0] * np.float32(N)).astype(jnp.int32)
        cond_j = jnp.logical_and(t_idx < kj, j <= Jr)
        M = jnp.minimum(M, jnp.where(cond_j, j, E))
    M = jnp.where(M == E, 0, M)
    m_ref[...] = M
    Mp = jnp.zeros((B, N), jnp.float32)
    for j in range(E):
        Mp = jnp.where(M == j, probs_ref[j], Mp)
    mp_ref[...] = Mp


def _fused_body(c_ref, x_ref, w_ref, m_ref, mp_ref, probs_ref, mask_ref):
    b = pl.program_id(0)
    t = pl.program_id(1)
    off = pl.multiple_of(t * TN, TN)
    w = w_ref[...]
    xb = x_ref[0]
    logits = jax.lax.dot_general(
        w, xb, (((1,), (1,)), ((), ())), preferred_element_type=jnp.float32
    )
    mx = jnp.max(logits, axis=0, keepdims=True)
    e = jnp.exp(logits - mx)
    p = e / jnp.sum(e, axis=0, keepdims=True)
    probs_ref[:, b, pl.ds(off, TN)] = p

    @pl.when(jnp.logical_and(b == B - 1, t == NT - 1))
    def _():
        _route(c_ref, probs_ref, m_ref, mp_ref, mask_ref)


def kernel(x, c, W):
    M, Mp = pl.pallas_call(
        _fused_body,
        grid=(B, NT),
        in_specs=[
            pl.BlockSpec(memory_space=pltpu.MemorySpace.SMEM),
            pl.BlockSpec((1, TN, D), lambda b, t: (b, t, 0)),
            pl.BlockSpec((E, D), lambda b, t: (0, 0)),
        ],
        out_specs=[
            pl.BlockSpec((B, N), lambda b, t: (0, 0)),
            pl.BlockSpec((B, N), lambda b, t: (0, 0)),
        ],
        out_shape=[
            jax.ShapeDtypeStruct((B, N), jnp.int32),
            jax.ShapeDtypeStruct((B, N), jnp.float32),
        ],
        scratch_shapes=[
            pltpu.VMEM((E, B, N), jnp.float32),
            pltpu.VMEM((B, N), jnp.int32),
        ],
    )(c.reshape(E, 1), x, W)
    return M, Mp
